# baseline (speedup 1.0000x reference)
import functools

import jax
import jax.numpy as jnp
from jax import lax
from jax.experimental import pallas as pl
from jax.experimental.pallas import tpu as pltpu
from jax.experimental.pallas import tpu_sc as plsc

NC = 2
NS = 16
L = 16
UNROLL = 4


def _sc_store(mem_hbm, tid_hbm, idx_hbm, val_hbm, ntid_hbm,
              out_hbm, otid_hbm,
              idx_v, ntid_v, win_v, rows_v, jlist_v, keep_v, tid_v, mbuf_v,
              isem, i2sem, nsem, gsem, g2sem, ssem):
  M, D = mem_hbm.shape
  B = idx_hbm.shape[0]
  NW = NC * NS
  R = L * ((M + L * NW - 1) // (L * NW))
  NV = R // L
  B2 = B // 2

  w = lax.axis_index("c") * NS + lax.axis_index("s")
  base = jnp.minimum(w * R, M - R)
  lane = lax.iota(jnp.int32, L)

  cp_idx = pltpu.async_copy(idx_hbm.at[pl.ds(0, B2)],
                            idx_v.at[pl.ds(0, B2)], isem)
  cp_idx2 = pltpu.async_copy(idx_hbm.at[pl.ds(B2, B2)],
                             idx_v.at[pl.ds(B2, B2)], i2sem)
  cp_nt = pltpu.async_copy(ntid_hbm, ntid_v, nsem)
  cp_tid = pltpu.async_copy(tid_hbm.at[pl.ds(base, R)], tid_v, nsem)

  for i in range(NV):
    win_v[pl.ds(i * L, L)] = jnp.full((L,), -1, jnp.int32)

  def phase_a(cc, carry):
    ivs, lasts = [], []
    for u in range(UNROLL):
      iv = idx_v[pl.ds((cc * UNROLL + u) * L, L)]
      ivs.append(iv)
    for u in range(UNROLL):
      _, last = plsc.scan_count(ivs[u])
      lasts.append(last)
    for u in range(UNROLL):
      iv = ivs[u]
      keep = lasts[u] & (iv >= base) & (iv < base + R)
      loc = jnp.where(keep, iv - base, 0)
      plsc.store_scatter(win_v, [loc], (cc * UNROLL + u) * L + lane,
                         mask=keep)
    return carry

  HALF_CC = B2 // (L * UNROLL)
  with jax.named_scope("phase_a"):
    cp_idx.wait()
    lax.fori_loop(0, HALF_CC, phase_a, 0)
    cp_idx2.wait()
    lax.fori_loop(HALF_CC, 2 * HALF_CC, phase_a, 0)

  with jax.named_scope("tid_resolve"):
    cp_nt.wait()
    cp_tid.wait()
    for i in range(NV):
      wv = win_v[pl.ds(i * L, L)]
      have = wv >= 0
      nv = plsc.load_gather(ntid_v, [jnp.where(have, wv, 0)], mask=have)
      tid_v[pl.ds(i * L, L)] = jnp.where(have, nv, tid_v[pl.ds(i * L, L)])
    cp_otid = pltpu.async_copy(tid_v, otid_hbm.at[pl.ds(base, R)], nsem)

  def phase_c(i, counts):
    cnt, cnt2 = counts
    wv = win_v[pl.ds(i * L, L)]
    rowv = base + i * L + lane
    have = wv >= 0
    keep = (~have) & (rowv < M)
    plsc.store_compressed(rows_v.at[pl.ds(cnt, L)], rowv, mask=have)
    plsc.store_compressed(jlist_v.at[pl.ds(cnt, L)], wv, mask=have)
    plsc.store_compressed(keep_v.at[pl.ds(cnt2, L)], rowv, mask=keep)
    npc = plsc.all_reduce_population_count(have)
    npc2 = plsc.all_reduce_population_count(keep)
    return (cnt + lax.reduce_max(npc, (0,)),
            cnt2 + lax.reduce_max(npc2, (0,)))

  with jax.named_scope("phase_c"):
    cnt, cnt2 = lax.fori_loop(0, NV, phase_c, (0, 0))
  ncw = (cnt + L - 1) // L
  nck = (cnt2 + L - 1) // L

  def pad_winner(c, carry):
    jv = jlist_v[pl.ds(c * L, L)]
    rv = rows_v[pl.ds(c * L, L)]
    valid = (c * L + lane) < cnt
    pad = plsc.cummax(jnp.where(valid, rv * 8192 + jv, -1))
    jlist_v[pl.ds(c * L, L)] = jnp.where(
        valid, jv, lax.bitwise_and(pad, 8191))
    rows_v[pl.ds(c * L, L)] = jnp.where(
        valid, rv, lax.shift_right_logical(pad, 13))
    return carry

  def pad_keeper(c, carry):
    rv = keep_v[pl.ds(c * L, L)]
    valid = (c * L + lane) < cnt2
    keep_v[pl.ds(c * L, L)] = jnp.where(
        valid, rv, plsc.cummax(jnp.where(valid, rv, -1)))
    return carry

  lax.fori_loop(jnp.maximum(ncw - 1, 0), ncw, pad_winner, 0)
  lax.fori_loop(jnp.maximum(nck - 1, 0), nck, pad_keeper, 0)

  kb = ncw * L

  def fire_wg(c, carry, *, sem):
    jv = jlist_v[pl.ds(c * L, L)]
    pltpu.async_copy(val_hbm.at[jv], mbuf_v.at[pl.ds(c * L, L)], sem)
    return carry

  def drain_wg(c, carry, *, sem):
    jv = jlist_v[pl.ds(c * L, L)]
    pltpu.make_async_copy(
        val_hbm.at[jv], mbuf_v.at[pl.ds(c * L, L)], sem).wait()
    return carry

  def fire_kg(c, carry, *, sem):
    rv = keep_v[pl.ds(c * L, L)]
    pltpu.async_copy(mem_hbm.at[rv], mbuf_v.at[pl.ds(kb + c * L, L)], sem)
    return carry

  def drain_kg(c, carry, *, sem):
    rv = keep_v[pl.ds(c * L, L)]
    pltpu.make_async_copy(
        mem_hbm.at[rv], mbuf_v.at[pl.ds(kb + c * L, L)], sem).wait()
    return carry

  def fire_ws(c, carry):
    rv = rows_v[pl.ds(c * L, L)]
    pltpu.async_copy(mbuf_v.at[pl.ds(c * L, L)], out_hbm.at[rv], ssem)
    return carry

  def fire_ks(c, carry):
    rv = keep_v[pl.ds(c * L, L)]
    pltpu.async_copy(mbuf_v.at[pl.ds(kb + c * L, L)], out_hbm.at[rv], ssem)
    return carry

  def drain_ws(c, carry):
    rv = rows_v[pl.ds(c * L, L)]
    pltpu.make_async_copy(
        mbuf_v.at[pl.ds(c * L, L)], out_hbm.at[rv], ssem).wait()
    return carry

  def drain_ks(c, carry):
    rv = keep_v[pl.ds(c * L, L)]
    pltpu.make_async_copy(
        mbuf_v.at[pl.ds(kb + c * L, L)], out_hbm.at[rv], ssem).wait()
    return carry

  wh = ncw // 2
  kh = nck // 2
  with jax.named_scope("phase_d"):
    lax.fori_loop(0, wh, functools.partial(fire_wg, sem=gsem), 0)
    lax.fori_loop(wh, ncw, functools.partial(fire_wg, sem=g2sem), 0)
    lax.fori_loop(0, kh, functools.partial(fire_kg, sem=isem), 0)
    lax.fori_loop(kh, nck, functools.partial(fire_kg, sem=i2sem), 0)
    lax.fori_loop(0, wh, functools.partial(drain_wg, sem=gsem), 0)
    lax.fori_loop(0, wh, fire_ws, 0)
    lax.fori_loop(wh, ncw, functools.partial(drain_wg, sem=g2sem), 0)
    lax.fori_loop(wh, ncw, fire_ws, 0)
    lax.fori_loop(0, kh, functools.partial(drain_kg, sem=isem), 0)
    lax.fori_loop(0, kh, fire_ks, 0)
    lax.fori_loop(kh, nck, functools.partial(drain_kg, sem=i2sem), 0)
    lax.fori_loop(kh, nck, fire_ks, 0)
    lax.fori_loop(0, ncw, drain_ws, 0)
    lax.fori_loop(0, nck, drain_ks, 0)
    pltpu.make_async_copy(tid_v, otid_hbm.at[pl.ds(base, R)], nsem).wait()
    del cp_otid


@jax.jit
def kernel(mem, task_ids, idx, val, new_task_ids):
  M, D = mem.shape
  B = idx.shape[0]
  NW = NC * NS
  R = L * ((M + L * NW - 1) // (L * NW))

  mesh = plsc.VectorSubcoreMesh(
      core_axis_name="c", subcore_axis_name="s", num_cores=NC,
      num_subcores=NS)
  f = pl.kernel(
      _sc_store,
      out_type=(
          jax.ShapeDtypeStruct((M, D), jnp.float32),
          jax.ShapeDtypeStruct((M,), jnp.int32),
      ),
      mesh=mesh,
      compiler_params=pltpu.CompilerParams(
          needs_layout_passes=False,
          skip_device_barrier=True,
          disable_bounds_checks=True,
          disable_semaphore_checks=True,
      ),
      scratch_types=[
          pltpu.VMEM((B,), jnp.int32),
          pltpu.VMEM((B,), jnp.int32),
          pltpu.VMEM((R,), jnp.int32),
          pltpu.VMEM((R + L,), jnp.int32),
          pltpu.VMEM((R + L,), jnp.int32),
          pltpu.VMEM((R + L,), jnp.int32),
          pltpu.VMEM((R,), jnp.int32),
          pltpu.VMEM((R + 2 * L, D), jnp.float32),
          pltpu.SemaphoreType.DMA,
          pltpu.SemaphoreType.DMA,
          pltpu.SemaphoreType.DMA,
          pltpu.SemaphoreType.DMA,
          pltpu.SemaphoreType.DMA,
          pltpu.SemaphoreType.DMA,
      ],
  )
  return f(mem, task_ids, idx, val, new_task_ids)


# speedup vs baseline: 1.0189x; 1.0189x over previous
import functools

import jax
import jax.numpy as jnp
from jax import lax
from jax.experimental import pallas as pl
from jax.experimental.pallas import tpu as pltpu
from jax.experimental.pallas import tpu_sc as plsc

NC = 2
NS = 16
L = 16
UNROLL = 8


def _sc_store(mem_hbm, tid_hbm, idx_hbm, val_hbm, ntid_hbm,
              out_hbm, otid_hbm,
              idx_v, ntid_v, win_v, rows_v, jlist_v, keep_v, tid_v, mbuf_v,
              isem, i2sem, nsem, gsem, g2sem, ssem):
  M, D = mem_hbm.shape
  B = idx_hbm.shape[0]
  NW = NC * NS
  R = L * ((M + L * NW - 1) // (L * NW))
  NV = R // L
  B2 = B // 2

  w = lax.axis_index("c") * NS + lax.axis_index("s")
  base = jnp.minimum(w * R, M - R)
  lane = lax.iota(jnp.int32, L)

  cp_idx = pltpu.async_copy(idx_hbm.at[pl.ds(0, B2)],
                            idx_v.at[pl.ds(0, B2)], isem)
  cp_idx2 = pltpu.async_copy(idx_hbm.at[pl.ds(B2, B2)],
                             idx_v.at[pl.ds(B2, B2)], i2sem)
  cp_nt = pltpu.async_copy(ntid_hbm, ntid_v, nsem)
  cp_tid = pltpu.async_copy(tid_hbm.at[pl.ds(base, R)], tid_v, nsem)

  for i in range(NV):
    win_v[pl.ds(i * L, L)] = jnp.full((L,), -1, jnp.int32)

  def phase_a(cc, carry):
    ivs, lasts = [], []
    for u in range(UNROLL):
      iv = idx_v[pl.ds((cc * UNROLL + u) * L, L)]
      ivs.append(iv)
    for u in range(UNROLL):
      _, last = plsc.scan_count(ivs[u])
      lasts.append(last)
    for u in range(UNROLL):
      iv = ivs[u]
      keep = lasts[u] & (iv >= base) & (iv < base + R)
      loc = jnp.where(keep, iv - base, 0)
      plsc.store_scatter(win_v, [loc], (cc * UNROLL + u) * L + lane,
                         mask=keep)
    return carry

  HALF_CC = B2 // (L * UNROLL)
  with jax.named_scope("phase_a"):
    cp_idx.wait()
    lax.fori_loop(0, HALF_CC, phase_a, 0)
    cp_idx2.wait()
    lax.fori_loop(HALF_CC, 2 * HALF_CC, phase_a, 0)

  with jax.named_scope("tid_resolve"):
    cp_nt.wait()
    cp_tid.wait()
    for i in range(NV):
      wv = win_v[pl.ds(i * L, L)]
      have = wv >= 0
      nv = plsc.load_gather(ntid_v, [jnp.where(have, wv, 0)], mask=have)
      tid_v[pl.ds(i * L, L)] = jnp.where(have, nv, tid_v[pl.ds(i * L, L)])
    cp_otid = pltpu.async_copy(tid_v, otid_hbm.at[pl.ds(base, R)], nsem)

  def phase_c(i, counts):
    cnt, cnt2 = counts
    wv = win_v[pl.ds(i * L, L)]
    rowv = base + i * L + lane
    have = wv >= 0
    keep = (~have) & (rowv < M)
    plsc.store_compressed(rows_v.at[pl.ds(cnt, L)], rowv, mask=have)
    plsc.store_compressed(jlist_v.at[pl.ds(cnt, L)], wv, mask=have)
    plsc.store_compressed(keep_v.at[pl.ds(cnt2, L)], rowv, mask=keep)
    npc = plsc.all_reduce_population_count(have)
    npc2 = plsc.all_reduce_population_count(keep)
    return (cnt + lax.reduce_max(npc, (0,)),
            cnt2 + lax.reduce_max(npc2, (0,)))

  with jax.named_scope("phase_c"):
    cnt, cnt2 = lax.fori_loop(0, NV, phase_c, (0, 0))
  ncw = (cnt + L - 1) // L
  nck = (cnt2 + L - 1) // L

  def pad_winner(c, carry):
    jv = jlist_v[pl.ds(c * L, L)]
    rv = rows_v[pl.ds(c * L, L)]
    valid = (c * L + lane) < cnt
    pad = plsc.cummax(jnp.where(valid, rv * 8192 + jv, -1))
    jlist_v[pl.ds(c * L, L)] = jnp.where(
        valid, jv, lax.bitwise_and(pad, 8191))
    rows_v[pl.ds(c * L, L)] = jnp.where(
        valid, rv, lax.shift_right_logical(pad, 13))
    return carry

  def pad_keeper(c, carry):
    rv = keep_v[pl.ds(c * L, L)]
    valid = (c * L + lane) < cnt2
    keep_v[pl.ds(c * L, L)] = jnp.where(
        valid, rv, plsc.cummax(jnp.where(valid, rv, -1)))
    return carry

  lax.fori_loop(jnp.maximum(ncw - 1, 0), ncw, pad_winner, 0)
  lax.fori_loop(jnp.maximum(nck - 1, 0), nck, pad_keeper, 0)

  kb = ncw * L

  def fire_wg(c, carry, *, sem):
    jv = jlist_v[pl.ds(c * L, L)]
    pltpu.async_copy(val_hbm.at[jv], mbuf_v.at[pl.ds(c * L, L)], sem)
    return carry

  def drain_wg(c, carry, *, sem):
    jv = jlist_v[pl.ds(c * L, L)]
    pltpu.make_async_copy(
        val_hbm.at[jv], mbuf_v.at[pl.ds(c * L, L)], sem).wait()
    return carry

  def fire_kg(c, carry, *, sem):
    rv = keep_v[pl.ds(c * L, L)]
    pltpu.async_copy(mem_hbm.at[rv], mbuf_v.at[pl.ds(kb + c * L, L)], sem)
    return carry

  def drain_kg(c, carry, *, sem):
    rv = keep_v[pl.ds(c * L, L)]
    pltpu.make_async_copy(
        mem_hbm.at[rv], mbuf_v.at[pl.ds(kb + c * L, L)], sem).wait()
    return carry

  def fire_ws(c, carry):
    rv = rows_v[pl.ds(c * L, L)]
    pltpu.async_copy(mbuf_v.at[pl.ds(c * L, L)], out_hbm.at[rv], ssem)
    return carry

  def fire_ks(c, carry):
    rv = keep_v[pl.ds(c * L, L)]
    pltpu.async_copy(mbuf_v.at[pl.ds(kb + c * L, L)], out_hbm.at[rv], ssem)
    return carry

  def drain_ws(c, carry):
    rv = rows_v[pl.ds(c * L, L)]
    pltpu.make_async_copy(
        mbuf_v.at[pl.ds(c * L, L)], out_hbm.at[rv], ssem).wait()
    return carry

  def drain_ks(c, carry):
    rv = keep_v[pl.ds(c * L, L)]
    pltpu.make_async_copy(
        mbuf_v.at[pl.ds(kb + c * L, L)], out_hbm.at[rv], ssem).wait()
    return carry

  with jax.named_scope("phase_d"):
    lax.fori_loop(0, ncw, functools.partial(fire_wg, sem=gsem), 0)
    lax.fori_loop(0, nck, functools.partial(fire_kg, sem=g2sem), 0)
    lax.fori_loop(0, ncw, functools.partial(drain_wg, sem=gsem), 0)
    lax.fori_loop(0, ncw, fire_ws, 0)
    lax.fori_loop(0, nck, functools.partial(drain_kg, sem=g2sem), 0)
    lax.fori_loop(0, nck, fire_ks, 0)
    lax.fori_loop(0, ncw, drain_ws, 0)
    lax.fori_loop(0, nck, drain_ks, 0)
    pltpu.make_async_copy(tid_v, otid_hbm.at[pl.ds(base, R)], nsem).wait()
    del cp_otid


@jax.jit
def kernel(mem, task_ids, idx, val, new_task_ids):
  M, D = mem.shape
  B = idx.shape[0]
  NW = NC * NS
  R = L * ((M + L * NW - 1) // (L * NW))

  mesh = plsc.VectorSubcoreMesh(
      core_axis_name="c", subcore_axis_name="s", num_cores=NC,
      num_subcores=NS)
  f = pl.kernel(
      _sc_store,
      out_type=(
          jax.ShapeDtypeStruct((M, D), jnp.float32),
          jax.ShapeDtypeStruct((M,), jnp.int32),
      ),
      mesh=mesh,
      compiler_params=pltpu.CompilerParams(
          needs_layout_passes=False,
          skip_device_barrier=True,
          disable_bounds_checks=True,
          disable_semaphore_checks=True,
      ),
      scratch_types=[
          pltpu.VMEM((B,), jnp.int32),
          pltpu.VMEM((B,), jnp.int32),
          pltpu.VMEM((R,), jnp.int32),
          pltpu.VMEM((R + L,), jnp.int32),
          pltpu.VMEM((R + L,), jnp.int32),
          pltpu.VMEM((R + L,), jnp.int32),
          pltpu.VMEM((R,), jnp.int32),
          pltpu.VMEM((R + 2 * L, D), jnp.float32),
          pltpu.SemaphoreType.DMA,
          pltpu.SemaphoreType.DMA,
          pltpu.SemaphoreType.DMA,
          pltpu.SemaphoreType.DMA,
          pltpu.SemaphoreType.DMA,
          pltpu.SemaphoreType.DMA,
      ],
  )
  return f(mem, task_ids, idx, val, new_task_ids)


# speedup vs baseline: 1.0249x; 1.0058x over previous
import functools

import jax
import jax.numpy as jnp
from jax import lax
from jax.experimental import pallas as pl
from jax.experimental.pallas import tpu as pltpu
from jax.experimental.pallas import tpu_sc as plsc

NC = 2
NS = 16
L = 16
UNROLL = 8


def _sc_store(mem_hbm, tid_hbm, idx_hbm, val_hbm, ntid_hbm,
              out_hbm, otid_hbm,
              idx_v, ntid_v, win_v, rows_v, jlist_v, keep_v, tid_v, mbuf_v,
              isem, i2sem, nsem, gsem, g2sem, ssem):
  M, D = mem_hbm.shape
  B = idx_hbm.shape[0]
  NW = NC * NS
  R = L * ((M + L * NW - 1) // (L * NW))
  NV = R // L
  B2 = B // 2

  w = lax.axis_index("c") * NS + lax.axis_index("s")
  base = jnp.minimum(w * R, M - R)
  lane = lax.iota(jnp.int32, L)

  cp_idx = pltpu.async_copy(idx_hbm.at[pl.ds(0, B2)],
                            idx_v.at[pl.ds(0, B2)], isem)
  cp_idx2 = pltpu.async_copy(idx_hbm.at[pl.ds(B2, B2)],
                             idx_v.at[pl.ds(B2, B2)], i2sem)
  cp_nt = pltpu.async_copy(ntid_hbm, ntid_v, nsem)
  cp_tid = pltpu.async_copy(tid_hbm.at[pl.ds(base, R)], tid_v, nsem)

  for i in range(NV):
    win_v[pl.ds(i * L, L)] = jnp.full((L,), -1, jnp.int32)

  def phase_a(cc, carry):
    ivs, lasts = [], []
    for u in range(UNROLL):
      iv = idx_v[pl.ds((cc * UNROLL + u) * L, L)]
      ivs.append(iv)
    for u in range(UNROLL):
      _, last = plsc.scan_count(ivs[u])
      lasts.append(last)
    for u in range(UNROLL):
      iv = ivs[u]
      keep = lasts[u] & (iv >= base) & (iv < base + R)
      loc = jnp.where(keep, iv - base, 0)
      plsc.store_scatter(win_v, [loc], (cc * UNROLL + u) * L + lane,
                         mask=keep)
    return carry

  HALF_CC = B2 // (L * UNROLL)
  cp_idx.wait()
  lax.fori_loop(0, HALF_CC, phase_a, 0)
  cp_idx2.wait()
  lax.fori_loop(HALF_CC, 2 * HALF_CC, phase_a, 0)

  cp_nt.wait()
  cp_tid.wait()
  for i in range(NV):
    wv = win_v[pl.ds(i * L, L)]
    have = wv >= 0
    nv = plsc.load_gather(ntid_v, [jnp.where(have, wv, 0)], mask=have)
    tid_v[pl.ds(i * L, L)] = jnp.where(have, nv, tid_v[pl.ds(i * L, L)])
  cp_otid = pltpu.async_copy(tid_v, otid_hbm.at[pl.ds(base, R)], nsem)

  def phase_c(i, counts):
    cnt, cnt2 = counts
    wv = win_v[pl.ds(i * L, L)]
    rowv = base + i * L + lane
    have = wv >= 0
    keep = (~have) & (rowv < M)
    plsc.store_compressed(rows_v.at[pl.ds(cnt, L)], rowv, mask=have)
    plsc.store_compressed(jlist_v.at[pl.ds(cnt, L)], wv, mask=have)
    plsc.store_compressed(keep_v.at[pl.ds(cnt2, L)], rowv, mask=keep)
    npc = plsc.all_reduce_population_count(have)
    npc2 = plsc.all_reduce_population_count(keep)
    return (cnt + lax.reduce_max(npc, (0,)),
            cnt2 + lax.reduce_max(npc2, (0,)))

  cnt, cnt2 = lax.fori_loop(0, NV, phase_c, (0, 0))
  ncw = (cnt + L - 1) // L
  nck = (cnt2 + L - 1) // L

  def pad_winner(c, carry):
    jv = jlist_v[pl.ds(c * L, L)]
    rv = rows_v[pl.ds(c * L, L)]
    valid = (c * L + lane) < cnt
    pad = plsc.cummax(jnp.where(valid, rv * 8192 + jv, -1))
    jlist_v[pl.ds(c * L, L)] = jnp.where(
        valid, jv, lax.bitwise_and(pad, 8191))
    rows_v[pl.ds(c * L, L)] = jnp.where(
        valid, rv, lax.shift_right_logical(pad, 13))
    return carry

  def pad_keeper(c, carry):
    rv = keep_v[pl.ds(c * L, L)]
    valid = (c * L + lane) < cnt2
    keep_v[pl.ds(c * L, L)] = jnp.where(
        valid, rv, plsc.cummax(jnp.where(valid, rv, -1)))
    return carry

  lax.fori_loop(jnp.maximum(ncw - 1, 0), ncw, pad_winner, 0)
  lax.fori_loop(jnp.maximum(nck - 1, 0), nck, pad_keeper, 0)

  kb = ncw * L

  def fire_wg(c, carry, *, sem):
    jv = jlist_v[pl.ds(c * L, L)]
    pltpu.async_copy(val_hbm.at[jv], mbuf_v.at[pl.ds(c * L, L)], sem)
    return carry

  def drain_wg(c, carry, *, sem):
    jv = jlist_v[pl.ds(c * L, L)]
    pltpu.make_async_copy(
        val_hbm.at[jv], mbuf_v.at[pl.ds(c * L, L)], sem).wait()
    return carry

  def fire_kg(c, carry, *, sem):
    rv = keep_v[pl.ds(c * L, L)]
    pltpu.async_copy(mem_hbm.at[rv], mbuf_v.at[pl.ds(kb + c * L, L)], sem)
    return carry

  def drain_kg(c, carry, *, sem):
    rv = keep_v[pl.ds(c * L, L)]
    pltpu.make_async_copy(
        mem_hbm.at[rv], mbuf_v.at[pl.ds(kb + c * L, L)], sem).wait()
    return carry

  def fire_ws(c, carry):
    rv = rows_v[pl.ds(c * L, L)]
    pltpu.async_copy(mbuf_v.at[pl.ds(c * L, L)], out_hbm.at[rv], ssem)
    return carry

  def fire_ks(c, carry):
    rv = keep_v[pl.ds(c * L, L)]
    pltpu.async_copy(mbuf_v.at[pl.ds(kb + c * L, L)], out_hbm.at[rv], ssem)
    return carry

  def drain_ws(c, carry):
    rv = rows_v[pl.ds(c * L, L)]
    pltpu.make_async_copy(
        mbuf_v.at[pl.ds(c * L, L)], out_hbm.at[rv], ssem).wait()
    return carry

  def drain_ks(c, carry):
    rv = keep_v[pl.ds(c * L, L)]
    pltpu.make_async_copy(
        mbuf_v.at[pl.ds(kb + c * L, L)], out_hbm.at[rv], ssem).wait()
    return carry

  lax.fori_loop(0, ncw, functools.partial(fire_wg, sem=gsem), 0)
  lax.fori_loop(0, nck, functools.partial(fire_kg, sem=g2sem), 0)
  lax.fori_loop(0, ncw, functools.partial(drain_wg, sem=gsem), 0)
  lax.fori_loop(0, ncw, fire_ws, 0)
  lax.fori_loop(0, nck, functools.partial(drain_kg, sem=g2sem), 0)
  lax.fori_loop(0, nck, fire_ks, 0)
  lax.fori_loop(0, ncw, drain_ws, 0)
  lax.fori_loop(0, nck, drain_ks, 0)
  pltpu.make_async_copy(tid_v, otid_hbm.at[pl.ds(base, R)], nsem).wait()
  del cp_otid


@jax.jit
def kernel(mem, task_ids, idx, val, new_task_ids):
  M, D = mem.shape
  B = idx.shape[0]
  NW = NC * NS
  R = L * ((M + L * NW - 1) // (L * NW))

  mesh = plsc.VectorSubcoreMesh(
      core_axis_name="c", subcore_axis_name="s", num_cores=NC,
      num_subcores=NS)
  f = pl.kernel(
      _sc_store,
      out_type=(
          jax.ShapeDtypeStruct((M, D), jnp.float32),
          jax.ShapeDtypeStruct((M,), jnp.int32),
      ),
      mesh=mesh,
      compiler_params=pltpu.CompilerParams(
          needs_layout_passes=False,
          skip_device_barrier=True,
          disable_bounds_checks=True,
          disable_semaphore_checks=True,
      ),
      scratch_types=[
          pltpu.VMEM((B,), jnp.int32),
          pltpu.VMEM((B,), jnp.int32),
          pltpu.VMEM((R,), jnp.int32),
          pltpu.VMEM((R + L,), jnp.int32),
          pltpu.VMEM((R + L,), jnp.int32),
          pltpu.VMEM((R + L,), jnp.int32),
          pltpu.VMEM((R,), jnp.int32),
          pltpu.VMEM((R + 2 * L, D), jnp.float32),
          pltpu.SemaphoreType.DMA,
          pltpu.SemaphoreType.DMA,
          pltpu.SemaphoreType.DMA,
          pltpu.SemaphoreType.DMA,
          pltpu.SemaphoreType.DMA,
          pltpu.SemaphoreType.DMA,
      ],
  )
  return f(mem, task_ids, idx, val, new_task_ids)
